# baseline (speedup 1.0000x reference)
import jax
import jax.numpy as jnp
from jax import lax
from jax.experimental import pallas as pl
from jax.experimental.pallas import tpu as pltpu
from jax.experimental.pallas import tpu_sc as plsc

W = 8
D = 2 * W
L = 200
NC, NS, LANES = 2, 16, 16
NW = NC * NS
N = 4096 * L
NB = N // NW
SB = NB // L
CSEQ = 8
CS = CSEQ * L
G = NB // CS
HALF = G // 2

P5 = 0.030449004538683766
P4 = -0.1315818250887885
P3 = 0.28527268109058584
P2 = -0.4902307234234099
P1 = 0.9992354838332749
P0 = 9.975032552137188e-06


def _softplus(x):
    u = jnp.exp(-jnp.abs(x))
    p = ((((P5 * u + P4) * u + P3) * u + P2) * u + P1) * u + P0
    return jnp.maximum(x, 0.0) + p


def _body(ids_hbm, tab_hbm, out_hbm, idxall, ob0, ob1,
          sg0, sg1, so0, so1):
    wid = lax.axis_index("s") * NC + lax.axis_index("c")
    s_base = wid * SB
    iota = lax.iota(jnp.int32, LANES)
    fmask = iota < W
    pltpu.sync_copy(ids_hbm.at[pl.ds(s_base, SB)], idxall)

    def gather(g, ob, sg):
        for j in range(CSEQ):
            pltpu.async_copy(
                tab_hbm.at[idxall.at[g * CSEQ + j]],
                ob.at[j], sg)

    def drain_gather(g, ob, sg):
        for j in range(CSEQ):
            pltpu.make_async_copy(
                tab_hbm.at[idxall.at[g * CSEQ + j]],
                ob.at[j], sg).wait()

    def owin(g):
        return out_hbm.at[pl.ds(s_base + g * CSEQ, CSEQ)]

    def compute(ob):
        for j in range(CSEQ):
            def rows(i, _, j=j):
                x0 = ob[j, 2 * i, :]
                x1 = ob[j, 2 * i + 1, :]
                ob[j, 2 * i, :] = jnp.where(fmask, _softplus(x0), x0)
                ob[j, 2 * i + 1, :] = jnp.where(fmask, _softplus(x1), x1)
                return 0
            lax.fori_loop(0, L // 2, rows, 0)

    gather(0, ob0, sg0)

    def step(p, _):
        g0 = 2 * p

        @pl.when(p > 0)
        def _():
            pltpu.make_async_copy(ob1, owin(g0 - 1), so1).wait()

        gather(g0 + 1, ob1, sg1)
        drain_gather(g0, ob0, sg0)
        compute(ob0)
        pltpu.async_copy(ob0, owin(g0), so0)

        @pl.when(p < HALF - 1)
        def _():
            pltpu.make_async_copy(ob0, owin(g0), so0).wait()
            gather(g0 + 2, ob0, sg0)

        drain_gather(g0 + 1, ob1, sg1)
        compute(ob1)
        pltpu.async_copy(ob1, owin(g0 + 1), so1)
        return 0

    lax.fori_loop(0, HALF, step, 0)
    pltpu.make_async_copy(ob0, owin(G - 2), so0).wait()
    pltpu.make_async_copy(ob1, owin(G - 1), so1).wait()


@jax.jit
def kernel(token_ids, frequencies, amplitudes):
    B, LL = token_ids.shape
    tab = jnp.concatenate([frequencies, amplitudes], axis=1)
    mesh = plsc.VectorSubcoreMesh(core_axis_name="c", subcore_axis_name="s",
                                  num_cores=NC, num_subcores=NS)
    out = pl.kernel(
        _body,
        out_type=jax.ShapeDtypeStruct((B, LL, D), jnp.float32),
        mesh=mesh,
        scratch_types=[
            pltpu.VMEM((SB, L), jnp.int32),
            pltpu.VMEM((CSEQ, L, D), jnp.float32),
            pltpu.VMEM((CSEQ, L, D), jnp.float32),
            pltpu.SemaphoreType.DMA,
            pltpu.SemaphoreType.DMA,
            pltpu.SemaphoreType.DMA,
            pltpu.SemaphoreType.DMA,
        ],
        compiler_params=pltpu.CompilerParams(use_tc_tiling_on_sc=False),
    )(token_ids, tab)
    return out


# speedup vs baseline: 1.2202x; 1.2202x over previous
import jax
import jax.numpy as jnp
from jax import lax
from jax.experimental import pallas as pl
from jax.experimental.pallas import tpu as pltpu
from jax.experimental.pallas import tpu_sc as plsc

W = 8
D = 2 * W
L = 200
NC, NS, LANES = 2, 16, 16
NW = NC * NS
N = 4096 * L
NB = N // NW
SB = NB // L
CSEQ = 8
CS = CSEQ * L
G = NB // CS
HALF = G // 2

P5 = 0.030449004538683766
P4 = -0.1315818250887885
P3 = 0.28527268109058584
P2 = -0.4902307234234099
P1 = 0.9992354838332749
P0 = 9.975032552137188e-06


def _softplus(x):
    u = jnp.exp(-jnp.abs(x))
    p = ((((P5 * u + P4) * u + P3) * u + P2) * u + P1) * u + P0
    return jnp.maximum(x, 0.0) + p


def _body(ids_hbm, tab_hbm, out_hbm, idxall, ob0, ob1,
          sg0, sg1, so0, so1):
    wid = lax.axis_index("s") * NC + lax.axis_index("c")
    s_base = wid * SB
    iota = lax.iota(jnp.int32, LANES)
    fmask = iota < W
    pltpu.sync_copy(ids_hbm.at[pl.ds(s_base, SB)], idxall)

    def gather(g, ob, sg):
        for j in range(CSEQ):
            pltpu.async_copy(
                tab_hbm.at[idxall.at[g * CSEQ + j]],
                ob.at[j], sg)

    def drain_gather(g, ob, sg):
        for j in range(CSEQ):
            pltpu.make_async_copy(
                tab_hbm.at[idxall.at[g * CSEQ + j]],
                ob.at[j], sg).wait()

    def owin(g):
        return out_hbm.at[pl.ds(s_base + g * CSEQ, CSEQ), :, pl.ds(0, D)]

    def compute(ob):
        for j in range(CSEQ):
            def rows(i, _, j=j):
                x0 = ob[j, 2 * i, :]
                x1 = ob[j, 2 * i + 1, :]
                ob[j, 2 * i, :] = jnp.where(fmask, _softplus(x0), x0)
                ob[j, 2 * i + 1, :] = jnp.where(fmask, _softplus(x1), x1)
                return 0
            lax.fori_loop(0, L // 2, rows, 0)

    gather(0, ob0, sg0)

    def step(p, _):
        g0 = 2 * p

        @pl.when(p > 0)
        def _():
            pltpu.make_async_copy(ob1, owin(g0 - 1), so1).wait()

        gather(g0 + 1, ob1, sg1)
        drain_gather(g0, ob0, sg0)
        compute(ob0)
        pltpu.async_copy(ob0, owin(g0), so0)

        @pl.when(p < HALF - 1)
        def _():
            pltpu.make_async_copy(ob0, owin(g0), so0).wait()
            gather(g0 + 2, ob0, sg0)

        drain_gather(g0 + 1, ob1, sg1)
        compute(ob1)
        pltpu.async_copy(ob1, owin(g0 + 1), so1)
        return 0

    lax.fori_loop(0, HALF, step, 0)
    pltpu.make_async_copy(ob0, owin(G - 2), so0).wait()
    pltpu.make_async_copy(ob1, owin(G - 1), so1).wait()


@jax.jit
def kernel(token_ids, frequencies, amplitudes):
    B, LL = token_ids.shape
    tab = jnp.concatenate([frequencies, amplitudes], axis=1)
    mesh = plsc.VectorSubcoreMesh(core_axis_name="c", subcore_axis_name="s",
                                  num_cores=NC, num_subcores=NS)
    out = pl.kernel(
        _body,
        out_type=jax.ShapeDtypeStruct((B, LL, 128), jnp.float32),
        mesh=mesh,
        scratch_types=[
            pltpu.VMEM((SB, L), jnp.int32),
            pltpu.VMEM((CSEQ, L, D), jnp.float32),
            pltpu.VMEM((CSEQ, L, D), jnp.float32),
            pltpu.SemaphoreType.DMA,
            pltpu.SemaphoreType.DMA,
            pltpu.SemaphoreType.DMA,
            pltpu.SemaphoreType.DMA,
        ],
        compiler_params=pltpu.CompilerParams(use_tc_tiling_on_sc=False),
    )(token_ids, tab)
    return out[:, :, :D]


# speedup vs baseline: 1.4238x; 1.1668x over previous
import jax
import jax.numpy as jnp
from jax import lax
from jax.experimental import pallas as pl
from jax.experimental.pallas import tpu as pltpu
from jax.experimental.pallas import tpu_sc as plsc

W = 8
D = 2 * W
L = 200
NC, NS, LANES = 2, 16, 16
NW = NC * NS
N = 4096 * L
NB = N // NW
SB = NB // L
CSEQ = 8
CS = CSEQ * L
G = NB // CS
HALF = G // 2

P5 = 0.030449004538683766
P4 = -0.1315818250887885
P3 = 0.28527268109058584
P2 = -0.4902307234234099
P1 = 0.9992354838332749
P0 = 9.975032552137188e-06


def _softplus(x):
    u = jnp.exp(-jnp.abs(x))
    p = ((((P5 * u + P4) * u + P3) * u + P2) * u + P1) * u + P0
    return jnp.maximum(x, 0.0) + p


BW = 8192
NBLK = (1000000 + BW - 1) // BW


def _prep_body(ft_ref, at_ref, o_ref):
    f = ft_ref[...]
    sp = jnp.maximum(f, 0.0) + jnp.log1p(jnp.exp(-jnp.abs(f)))
    x = jnp.concatenate([sp, at_ref[...]], axis=0)
    y = jnp.swapaxes(x, 0, 1).reshape(BW // 8, 8, D)
    o_ref[...] = jnp.concatenate([y[:, j, :] for j in range(8)], axis=1)


def _prep_table(frequencies, amplitudes):
    V = frequencies.shape[0]
    out = pl.pallas_call(
        _prep_body,
        grid=(NBLK,),
        in_specs=[
            pl.BlockSpec((8, BW), lambda i: (0, i)),
            pl.BlockSpec((8, BW), lambda i: (0, i)),
        ],
        out_specs=pl.BlockSpec((BW // 8, 128), lambda i: (i, 0)),
        out_shape=jax.ShapeDtypeStruct((V * D // 128, 128), jnp.float32),
    )(frequencies.T, amplitudes.T)
    return out.reshape(V * D).reshape(V, D)


def _body(ids_hbm, tab_hbm, out_hbm, idxall, ob0, ob1,
          sg0, sg1, so0, so1):
    wid = lax.axis_index("s") * NC + lax.axis_index("c")
    s_base = wid * SB
    iota = lax.iota(jnp.int32, LANES)
    fmask = iota < W
    pltpu.sync_copy(ids_hbm.at[pl.ds(s_base, SB)], idxall)

    def gather(g, ob, sg):
        for j in range(CSEQ):
            pltpu.async_copy(
                tab_hbm.at[idxall.at[g * CSEQ + j]],
                ob.at[j], sg)

    def drain_gather(g, ob, sg):
        for j in range(CSEQ):
            pltpu.make_async_copy(
                tab_hbm.at[idxall.at[g * CSEQ + j]],
                ob.at[j], sg).wait()

    def owin(g):
        return out_hbm.at[pl.ds(s_base + g * CSEQ, CSEQ), :, pl.ds(0, D)]

    def compute(ob):
        for j in range(CSEQ):
            def rows(i, _, j=j):
                x0 = ob[j, 2 * i, :]
                x1 = ob[j, 2 * i + 1, :]
                ob[j, 2 * i, :] = jnp.where(fmask, _softplus(x0), x0)
                ob[j, 2 * i + 1, :] = jnp.where(fmask, _softplus(x1), x1)
                return 0
            lax.fori_loop(0, L // 2, rows, 0)

    gather(0, ob0, sg0)

    def step(p, _):
        g0 = 2 * p

        @pl.when(p > 0)
        def _():
            pltpu.make_async_copy(ob1, owin(g0 - 1), so1).wait()

        gather(g0 + 1, ob1, sg1)
        drain_gather(g0, ob0, sg0)
        compute(ob0)
        pltpu.async_copy(ob0, owin(g0), so0)

        @pl.when(p < HALF - 1)
        def _():
            pltpu.make_async_copy(ob0, owin(g0), so0).wait()
            gather(g0 + 2, ob0, sg0)

        drain_gather(g0 + 1, ob1, sg1)
        compute(ob1)
        pltpu.async_copy(ob1, owin(g0 + 1), so1)
        return 0

    lax.fori_loop(0, HALF, step, 0)
    pltpu.make_async_copy(ob0, owin(G - 2), so0).wait()
    pltpu.make_async_copy(ob1, owin(G - 1), so1).wait()


@jax.jit
def kernel(token_ids, frequencies, amplitudes):
    B, LL = token_ids.shape
    tab = _prep_table(frequencies, amplitudes)
    mesh = plsc.VectorSubcoreMesh(core_axis_name="c", subcore_axis_name="s",
                                  num_cores=NC, num_subcores=NS)
    out = pl.kernel(
        _body,
        out_type=jax.ShapeDtypeStruct((B, LL, 128), jnp.float32),
        mesh=mesh,
        scratch_types=[
            pltpu.VMEM((SB, L), jnp.int32),
            pltpu.VMEM((CSEQ, L, D), jnp.float32),
            pltpu.VMEM((CSEQ, L, D), jnp.float32),
            pltpu.SemaphoreType.DMA,
            pltpu.SemaphoreType.DMA,
            pltpu.SemaphoreType.DMA,
            pltpu.SemaphoreType.DMA,
        ],
        compiler_params=pltpu.CompilerParams(use_tc_tiling_on_sc=False),
    )(token_ids, tab)
    return out[:, :, :D]


# speedup vs baseline: 2.1265x; 1.4936x over previous
import jax
import jax.numpy as jnp
from jax import lax
from jax.experimental import pallas as pl
from jax.experimental.pallas import tpu as pltpu
from jax.experimental.pallas import tpu_sc as plsc

W = 8
D = 2 * W
L = 200
NC, NS, LANES = 2, 16, 16
NW = NC * NS
N = 4096 * L
NB = N // NW
SB = NB // L
CSEQ = 8
CS = CSEQ * L
G = NB // CS
HALF = G // 2

P5 = 0.030449004538683766
P4 = -0.1315818250887885
P3 = 0.28527268109058584
P2 = -0.4902307234234099
P1 = 0.9992354838332749
P0 = 9.975032552137188e-06


def _softplus(x):
    u = jnp.exp(-jnp.abs(x))
    p = ((((P5 * u + P4) * u + P3) * u + P2) * u + P1) * u + P0
    return jnp.maximum(x, 0.0) + p


BW = 8192
NBLK = (1000000 + BW - 1) // BW


def _prep_body(ft_ref, at_ref, o_ref):
    f = ft_ref[...]
    sp = jnp.maximum(f, 0.0) + jnp.log1p(jnp.exp(-jnp.abs(f)))
    x = jnp.concatenate([sp, at_ref[...]], axis=0)
    y = jnp.swapaxes(x, 0, 1).reshape(BW // 8, 8, D)
    o_ref[...] = jnp.concatenate([y[:, j, :] for j in range(8)], axis=1)


def _prep_table(frequencies, amplitudes):
    V = frequencies.shape[0]
    out = pl.pallas_call(
        _prep_body,
        grid=(NBLK,),
        in_specs=[
            pl.BlockSpec((8, BW), lambda i: (0, i)),
            pl.BlockSpec((8, BW), lambda i: (0, i)),
        ],
        out_specs=pl.BlockSpec((BW // 8, 128), lambda i: (i, 0)),
        out_shape=jax.ShapeDtypeStruct((V * D // 128, 128), jnp.float32),
    )(frequencies.T, amplitudes.T)
    return out.reshape(V * D).reshape(V, D)


def _body(ids_hbm, tab_hbm, out_hbm, idxall, ob0, ob1,
          sg0, sg1, so0, so1):
    wid = lax.axis_index("s") * NC + lax.axis_index("c")
    s_base = wid * SB
    pltpu.sync_copy(ids_hbm.at[pl.ds(s_base, SB)], idxall)

    def gather(g, ob, sg):
        for j in range(CSEQ):
            pltpu.async_copy(
                tab_hbm.at[idxall.at[g * CSEQ + j]],
                ob.at[j], sg)

    def drain_gather(g, ob, sg):
        for j in range(CSEQ):
            pltpu.make_async_copy(
                tab_hbm.at[idxall.at[g * CSEQ + j]],
                ob.at[j], sg).wait()

    def owin(g):
        return out_hbm.at[pl.ds(s_base + g * CSEQ, CSEQ), :, pl.ds(0, D)]

    gather(0, ob0, sg0)

    def step(p, _):
        g0 = 2 * p

        @pl.when(p > 0)
        def _():
            pltpu.make_async_copy(ob1, owin(g0 - 1), so1).wait()

        gather(g0 + 1, ob1, sg1)
        drain_gather(g0, ob0, sg0)
        pltpu.async_copy(ob0, owin(g0), so0)

        @pl.when(p < HALF - 1)
        def _():
            pltpu.make_async_copy(ob0, owin(g0), so0).wait()
            gather(g0 + 2, ob0, sg0)

        drain_gather(g0 + 1, ob1, sg1)
        pltpu.async_copy(ob1, owin(g0 + 1), so1)
        return 0

    lax.fori_loop(0, HALF, step, 0)
    pltpu.make_async_copy(ob0, owin(G - 2), so0).wait()
    pltpu.make_async_copy(ob1, owin(G - 1), so1).wait()


@jax.jit
def kernel(token_ids, frequencies, amplitudes):
    B, LL = token_ids.shape
    tab = _prep_table(frequencies, amplitudes)
    mesh = plsc.VectorSubcoreMesh(core_axis_name="c", subcore_axis_name="s",
                                  num_cores=NC, num_subcores=NS)
    out = pl.kernel(
        _body,
        out_type=jax.ShapeDtypeStruct((B, LL, 128), jnp.float32),
        mesh=mesh,
        scratch_types=[
            pltpu.VMEM((SB, L), jnp.int32),
            pltpu.VMEM((CSEQ, L, D), jnp.float32),
            pltpu.VMEM((CSEQ, L, D), jnp.float32),
            pltpu.SemaphoreType.DMA,
            pltpu.SemaphoreType.DMA,
            pltpu.SemaphoreType.DMA,
            pltpu.SemaphoreType.DMA,
        ],
        compiler_params=pltpu.CompilerParams(use_tc_tiling_on_sc=False),
    )(token_ids, tab)
    return out[:, :, :D]


# speedup vs baseline: 2.1267x; 1.0001x over previous
import jax
import jax.numpy as jnp
from jax import lax
from jax.experimental import pallas as pl
from jax.experimental.pallas import tpu as pltpu
from jax.experimental.pallas import tpu_sc as plsc

W = 8
D = 2 * W
L = 200
NC, NS = 2, 16
NW = NC * NS
N = 4096 * L
NB = N // NW
SB = NB // L
CSEQ = 8
CS = CSEQ * L
G = NB // CS
HALF = G // 2

BW = 8192
NBLK = (1000000 + BW - 1) // BW


def _prep_body(ft_ref, at_ref, o_ref):
    f = ft_ref[...]
    sp = jnp.maximum(f, 0.0) + jnp.log1p(jnp.exp(-jnp.abs(f)))
    x = jnp.concatenate([sp, at_ref[...]], axis=0)
    y = jnp.swapaxes(x, 0, 1).reshape(BW // 8, 8, D)
    o_ref[...] = jnp.concatenate([y[:, j, :] for j in range(8)], axis=1)


def _prep_table(frequencies, amplitudes):
    V = frequencies.shape[0]
    out = pl.pallas_call(
        _prep_body,
        grid=(NBLK,),
        in_specs=[
            pl.BlockSpec((8, BW), lambda i: (0, i)),
            pl.BlockSpec((8, BW), lambda i: (0, i)),
        ],
        out_specs=pl.BlockSpec((BW // 8, 128), lambda i: (i, 0)),
        out_shape=jax.ShapeDtypeStruct((V * D // 128, 128), jnp.float32),
    )(frequencies.T, amplitudes.T)
    return out.reshape(V * D).reshape(V, D)


def _body(ids_hbm, tab_hbm, out_hbm, idxall, ob0, ob1,
          sg0, sg1, so0, so1):
    wid = lax.axis_index("s") * NC + lax.axis_index("c")
    s_base = wid * SB
    pltpu.sync_copy(ids_hbm.at[pl.ds(s_base, SB)], idxall)

    def gather(g, ob, sg):
        for j in range(CSEQ):
            pltpu.async_copy(
                tab_hbm.at[idxall.at[g * CSEQ + j]],
                ob.at[j], sg)

    def drain_gather(g, ob, sg):
        for j in range(CSEQ):
            pltpu.make_async_copy(
                tab_hbm.at[idxall.at[g * CSEQ + j]],
                ob.at[j], sg).wait()

    def owin(g):
        return out_hbm.at[pl.ds(s_base + g * CSEQ, CSEQ), :, pl.ds(0, D)]

    gather(0, ob0, sg0)

    def step(p, _):
        g0 = 2 * p

        @pl.when(p > 0)
        def _():
            pltpu.make_async_copy(ob1, owin(g0 - 1), so1).wait()

        gather(g0 + 1, ob1, sg1)
        drain_gather(g0, ob0, sg0)
        pltpu.async_copy(ob0, owin(g0), so0)

        @pl.when(p < HALF - 1)
        def _():
            pltpu.make_async_copy(ob0, owin(g0), so0).wait()
            gather(g0 + 2, ob0, sg0)

        drain_gather(g0 + 1, ob1, sg1)
        pltpu.async_copy(ob1, owin(g0 + 1), so1)
        return 0

    lax.fori_loop(0, HALF, step, 0)
    pltpu.make_async_copy(ob0, owin(G - 2), so0).wait()
    pltpu.make_async_copy(ob1, owin(G - 1), so1).wait()


@jax.jit
def kernel(token_ids, frequencies, amplitudes):
    B, LL = token_ids.shape
    tab = _prep_table(frequencies, amplitudes)
    mesh = plsc.VectorSubcoreMesh(core_axis_name="c", subcore_axis_name="s",
                                  num_cores=NC, num_subcores=NS)
    out = pl.kernel(
        _body,
        out_type=jax.ShapeDtypeStruct((B, LL, 128), jnp.float32),
        mesh=mesh,
        scratch_types=[
            pltpu.VMEM((SB, L), jnp.int32),
            pltpu.VMEM((CSEQ, L, D), jnp.float32),
            pltpu.VMEM((CSEQ, L, D), jnp.float32),
            pltpu.SemaphoreType.DMA,
            pltpu.SemaphoreType.DMA,
            pltpu.SemaphoreType.DMA,
            pltpu.SemaphoreType.DMA,
        ],
        compiler_params=pltpu.CompilerParams(use_tc_tiling_on_sc=False),
    )(token_ids, tab)
    return out[:, :, :D]
